# baseline (speedup 1.0000x reference)
import functools

import jax
import jax.numpy as jnp
from jax import lax
from jax.experimental import pallas as pl
from jax.experimental.pallas import tpu as pltpu
from jax.experimental.pallas import tpu_sc as plsc

BATCH = 16384
EMBED_DIM = 64
NUM_CORES = 2
NUM_SUBCORES = 16
NUM_WORKERS = NUM_CORES * NUM_SUBCORES
BPW = BATCH // NUM_WORKERS
CHUNK = 128
NCHUNK = BPW // CHUNK
LANES = 16
NGROUP = BPW // LANES
HALF_GROUPS = NGROUP // 2
UBLK = 512


def _relayout_body(t_ref, o_ref):
    z = t_ref[...]
    o_ref[:, 0:EMBED_DIM] = jnp.swapaxes(z[:, 0:UBLK // 2], 0, 1)
    o_ref[:, EMBED_DIM:2 * EMBED_DIM] = jnp.swapaxes(z[:, UBLK // 2:], 0, 1)


@functools.partial(jax.jit, static_argnames=("rows",))
def _relayout(table_t, rows):
    grid = (rows + UBLK - 1) // UBLK
    return pl.pallas_call(
        _relayout_body,
        out_shape=jax.ShapeDtypeStruct((grid * (UBLK // 2), 2 * EMBED_DIM),
                                       jnp.float32),
        grid=(grid,),
        in_specs=[pl.BlockSpec((EMBED_DIM, UBLK), lambda i: (0, i))],
        out_specs=pl.BlockSpec((UBLK // 2, 2 * EMBED_DIM), lambda i: (i, 0)),
    )(table_t)


def _sc_body(users_hbm, items_hbm, uemb_hbm, iemb_hbm, ub_hbm, ib_hbm,
             gb_hbm, out_hbm,
             uidx_v, iidx_v, uridx_v, iridx_v, urows_v, irows_v,
             ubias_v, ibias_v, gb_v, out_v, sem):
    wid = lax.axis_index("s") * NUM_CORES + lax.axis_index("c")
    base = wid * BPW

    for j in range(NCHUNK):
        pltpu.sync_copy(users_hbm.at[pl.ds(base + j * CHUNK, CHUNK)],
                        uidx_v.at[j])
        pltpu.sync_copy(items_hbm.at[pl.ds(base + j * CHUNK, CHUNK)],
                        iidx_v.at[j])
    pltpu.sync_copy(gb_hbm, gb_v)

    def ridx_body(t, _):
        j = t // (CHUNK // LANES)
        o = (t % (CHUNK // LANES)) * LANES
        u16 = uidx_v[j, pl.ds(o, LANES)]
        i16 = iidx_v[j, pl.ds(o, LANES)]
        uridx_v[j, pl.ds(o, LANES)] = ((u16 >> 9) << 8) | (u16 & 255)
        iridx_v[j, pl.ds(o, LANES)] = ((i16 >> 9) << 8) | (i16 & 255)
        return 0

    lax.fori_loop(0, NCHUNK * (CHUNK // LANES), ridx_body, 0)

    bias_copies = []
    for j in range(NCHUNK):
        sl = pl.ds(j * CHUNK, CHUNK)
        bias_copies.append(pltpu.async_copy(ub_hbm.at[uidx_v.at[j]],
                                            ubias_v.at[sl], sem))
        bias_copies.append(pltpu.async_copy(ib_hbm.at[iidx_v.at[j]],
                                            ibias_v.at[sl], sem))

    iota = jnp.arange(LANES, dtype=jnp.int32)
    gbias = gb_v[...]

    for h in range(2):
        copies = []
        for jj in range(NCHUNK // 2):
            j = h * (NCHUNK // 2) + jj
            sl = pl.ds(jj * CHUNK, CHUNK)
            copies.append(pltpu.async_copy(uemb_hbm.at[uridx_v.at[j]],
                                           urows_v.at[sl], sem))
            copies.append(pltpu.async_copy(iemb_hbm.at[iridx_v.at[j]],
                                           irows_v.at[sl], sem))
        for c in copies:
            c.wait()

        def group_body(g, _):
            rows = g * LANES + iota
            gb_off = h * (BPW // 2)
            j = (gb_off + g * LANES) // CHUNK
            o = (gb_off + g * LANES) % CHUNK
            ucol = ((uidx_v[j, pl.ds(o, LANES)] >> 8) & 1) << 6
            icol = ((iidx_v[j, pl.ds(o, LANES)] >> 8) & 1) << 6
            acc = jnp.zeros((LANES,), dtype=jnp.float32)
            for d in range(EMBED_DIM):
                cu = plsc.load_gather(urows_v, [rows, ucol + d])
                ci = plsc.load_gather(irows_v, [rows, icol + d])
                acc = acc + cu * ci
            ub = ubias_v[pl.ds(gb_off + g * LANES, LANES)]
            ib = ibias_v[pl.ds(gb_off + g * LANES, LANES)]
            out_v[pl.ds(gb_off + g * LANES, LANES)] = acc + ub + ib + gbias
            return 0

        lax.fori_loop(0, HALF_GROUPS, group_body, 0)

    for c in bias_copies:
        c.wait()
    pltpu.sync_copy(out_v, out_hbm.at[pl.ds(base, BPW)])


@jax.jit
def _bpr_scores(users, items, uemb2, iemb2, user_bias, item_bias, gb16):
    mesh = plsc.VectorSubcoreMesh(core_axis_name="c", subcore_axis_name="s",
                                  num_cores=NUM_CORES,
                                  num_subcores=NUM_SUBCORES)
    f = pl.kernel(
        _sc_body,
        out_type=jax.ShapeDtypeStruct((BATCH,), jnp.float32),
        mesh=mesh,
        compiler_params=pltpu.CompilerParams(needs_layout_passes=False,
                                             use_tc_tiling_on_sc=False),
        scratch_types=[
            pltpu.VMEM((NCHUNK, CHUNK), jnp.int32),
            pltpu.VMEM((NCHUNK, CHUNK), jnp.int32),
            pltpu.VMEM((NCHUNK, CHUNK), jnp.int32),
            pltpu.VMEM((NCHUNK, CHUNK), jnp.int32),
            pltpu.VMEM((BPW // 2, 2 * EMBED_DIM), jnp.float32),
            pltpu.VMEM((BPW // 2, 2 * EMBED_DIM), jnp.float32),
            pltpu.VMEM((BPW,), jnp.float32),
            pltpu.VMEM((BPW,), jnp.float32),
            pltpu.VMEM((LANES,), jnp.float32),
            pltpu.VMEM((BPW,), jnp.float32),
            pltpu.SemaphoreType.DMA,
        ],
    )
    return f(users, items, uemb2, iemb2, user_bias, item_bias, gb16)


def kernel(users, items, user_emb_w, item_emb_w, user_bias_w, item_bias_w,
           global_bias):
    users = users.astype(jnp.int32)
    items = items.astype(jnp.int32)
    uemb2 = _relayout(user_emb_w.T, rows=user_emb_w.shape[0])
    iemb2 = _relayout(item_emb_w.T, rows=item_emb_w.shape[0])
    gb16 = jnp.broadcast_to(global_bias.reshape(()), (16,))
    return _bpr_scores(users, items, uemb2, iemb2,
                       user_bias_w.reshape(-1), item_bias_w.reshape(-1),
                       gb16)


# speedup vs baseline: 1.8127x; 1.8127x over previous
import functools

import jax
import jax.numpy as jnp
from jax import lax
from jax.experimental import pallas as pl
from jax.experimental.pallas import tpu as pltpu
from jax.experimental.pallas import tpu_sc as plsc

BATCH = 16384
EMBED_DIM = 64
NUM_CORES = 2
NUM_SUBCORES = 16
NUM_WORKERS = NUM_CORES * NUM_SUBCORES
BPW = BATCH // NUM_WORKERS
CHUNK = 128
NCHUNK = BPW // CHUNK
LANES = 16
NGROUP = BPW // LANES
HALF_GROUPS = NGROUP // 2
UBLK = 2048
SEG_SH = 11
HALF = UBLK // 2
HALF_SH = SEG_SH - 1


def _relayout_body(t_ref, o_ref):
    z = t_ref[...]
    eye = jnp.eye(EMBED_DIM, dtype=jnp.float32)
    o_ref[:, 0:EMBED_DIM] = lax.dot_general(
        z[:, 0:UBLK // 2], eye, (((0,), (0,)), ((), ())),
        precision=lax.Precision.HIGHEST)
    o_ref[:, EMBED_DIM:2 * EMBED_DIM] = lax.dot_general(
        z[:, UBLK // 2:], eye, (((0,), (0,)), ((), ())),
        precision=lax.Precision.HIGHEST)


@functools.partial(jax.jit, static_argnames=("rows",))
def _relayout(table_t, rows):
    grid = (rows + UBLK - 1) // UBLK
    return pl.pallas_call(
        _relayout_body,
        out_shape=jax.ShapeDtypeStruct((grid * (UBLK // 2), 2 * EMBED_DIM),
                                       jnp.float32),
        grid=(grid,),
        in_specs=[pl.BlockSpec((EMBED_DIM, UBLK), lambda i: (0, i))],
        out_specs=pl.BlockSpec((UBLK // 2, 2 * EMBED_DIM), lambda i: (i, 0)),
    )(table_t)


def _sc_body(users_hbm, items_hbm, uemb_hbm, iemb_hbm, ub_hbm, ib_hbm,
             gb_hbm, out_hbm,
             uidx_v, iidx_v, uridx_v, iridx_v, urows_v, irows_v,
             ubias_v, ibias_v, gb_v, out_v, sem):
    wid = lax.axis_index("s") * NUM_CORES + lax.axis_index("c")
    base = wid * BPW

    for j in range(NCHUNK):
        pltpu.sync_copy(users_hbm.at[pl.ds(base + j * CHUNK, CHUNK)],
                        uidx_v.at[j])
        pltpu.sync_copy(items_hbm.at[pl.ds(base + j * CHUNK, CHUNK)],
                        iidx_v.at[j])
    pltpu.sync_copy(gb_hbm, gb_v)

    def ridx_body(t, _):
        j = t // (CHUNK // LANES)
        o = (t % (CHUNK // LANES)) * LANES
        u16 = uidx_v[j, pl.ds(o, LANES)]
        i16 = iidx_v[j, pl.ds(o, LANES)]
        uridx_v[j, pl.ds(o, LANES)] = \
            ((u16 >> SEG_SH) << HALF_SH) | (u16 & (HALF - 1))
        iridx_v[j, pl.ds(o, LANES)] = \
            ((i16 >> SEG_SH) << HALF_SH) | (i16 & (HALF - 1))
        return 0

    lax.fori_loop(0, NCHUNK * (CHUNK // LANES), ridx_body, 0)

    bias_copies = []
    for j in range(NCHUNK):
        sl = pl.ds(j * CHUNK, CHUNK)
        bias_copies.append(pltpu.async_copy(ub_hbm.at[uidx_v.at[j]],
                                            ubias_v.at[sl], sem))
        bias_copies.append(pltpu.async_copy(ib_hbm.at[iidx_v.at[j]],
                                            ibias_v.at[sl], sem))

    iota = jnp.arange(LANES, dtype=jnp.int32)
    gbias = gb_v[...]

    for h in range(2):
        copies = []
        for jj in range(NCHUNK // 2):
            j = h * (NCHUNK // 2) + jj
            sl = pl.ds(jj * CHUNK, CHUNK)
            copies.append(pltpu.async_copy(uemb_hbm.at[uridx_v.at[j]],
                                           urows_v.at[sl], sem))
            copies.append(pltpu.async_copy(iemb_hbm.at[iridx_v.at[j]],
                                           irows_v.at[sl], sem))
        for c in copies:
            c.wait()

        def group_body(g, _):
            rows = g * LANES + iota
            gb_off = h * (BPW // 2)
            j = (gb_off + g * LANES) // CHUNK
            o = (gb_off + g * LANES) % CHUNK
            ucol = ((uidx_v[j, pl.ds(o, LANES)] >> HALF_SH) & 1) << 6
            icol = ((iidx_v[j, pl.ds(o, LANES)] >> HALF_SH) & 1) << 6
            acc = jnp.zeros((LANES,), dtype=jnp.float32)
            for d in range(EMBED_DIM):
                cu = plsc.load_gather(urows_v, [rows, ucol + d])
                ci = plsc.load_gather(irows_v, [rows, icol + d])
                acc = acc + cu * ci
            ub = ubias_v[pl.ds(gb_off + g * LANES, LANES)]
            ib = ibias_v[pl.ds(gb_off + g * LANES, LANES)]
            out_v[pl.ds(gb_off + g * LANES, LANES)] = acc + ub + ib + gbias
            return 0

        lax.fori_loop(0, HALF_GROUPS, group_body, 0)

    for c in bias_copies:
        c.wait()
    pltpu.sync_copy(out_v, out_hbm.at[pl.ds(base, BPW)])


@jax.jit
def _bpr_scores(users, items, uemb2, iemb2, user_bias, item_bias, gb16):
    mesh = plsc.VectorSubcoreMesh(core_axis_name="c", subcore_axis_name="s",
                                  num_cores=NUM_CORES,
                                  num_subcores=NUM_SUBCORES)
    f = pl.kernel(
        _sc_body,
        out_type=jax.ShapeDtypeStruct((BATCH,), jnp.float32),
        mesh=mesh,
        compiler_params=pltpu.CompilerParams(needs_layout_passes=False,
                                             use_tc_tiling_on_sc=False),
        scratch_types=[
            pltpu.VMEM((NCHUNK, CHUNK), jnp.int32),
            pltpu.VMEM((NCHUNK, CHUNK), jnp.int32),
            pltpu.VMEM((NCHUNK, CHUNK), jnp.int32),
            pltpu.VMEM((NCHUNK, CHUNK), jnp.int32),
            pltpu.VMEM((BPW // 2, 2 * EMBED_DIM), jnp.float32),
            pltpu.VMEM((BPW // 2, 2 * EMBED_DIM), jnp.float32),
            pltpu.VMEM((BPW,), jnp.float32),
            pltpu.VMEM((BPW,), jnp.float32),
            pltpu.VMEM((LANES,), jnp.float32),
            pltpu.VMEM((BPW,), jnp.float32),
            pltpu.SemaphoreType.DMA,
        ],
    )
    return f(users, items, uemb2, iemb2, user_bias, item_bias, gb16)


def kernel(users, items, user_emb_w, item_emb_w, user_bias_w, item_bias_w,
           global_bias):
    users = users.astype(jnp.int32)
    items = items.astype(jnp.int32)
    uemb2 = _relayout(user_emb_w.T, rows=user_emb_w.shape[0])
    iemb2 = _relayout(item_emb_w.T, rows=item_emb_w.shape[0])
    gb16 = jnp.broadcast_to(global_bias.reshape(()), (16,))
    return _bpr_scores(users, items, uemb2, iemb2,
                       user_bias_w.reshape(-1), item_bias_w.reshape(-1),
                       gb16)


# speedup vs baseline: 1.8149x; 1.0012x over previous
import functools

import jax
import jax.numpy as jnp
from jax import lax
from jax.experimental import pallas as pl
from jax.experimental.pallas import tpu as pltpu
from jax.experimental.pallas import tpu_sc as plsc

BATCH = 16384
EMBED_DIM = 64
NUM_CORES = 2
NUM_SUBCORES = 16
NUM_WORKERS = NUM_CORES * NUM_SUBCORES
BPW = BATCH // NUM_WORKERS
CHUNK = 128
NCHUNK = BPW // CHUNK
LANES = 16
NGROUP = BPW // LANES
HALF_GROUPS = NGROUP // 2
UBLK = 2048
SEG_SH = 11
HALF = UBLK // 2
HALF_SH = SEG_SH - 1


def _relayout_body(t_ref, o_ref):
    z = t_ref[...]
    eye = jnp.eye(EMBED_DIM, dtype=jnp.float32)
    o_ref[:, 0:EMBED_DIM] = lax.dot_general(
        z[:, 0:UBLK // 2], eye, (((0,), (0,)), ((), ())),
        precision=lax.Precision.HIGHEST)
    o_ref[:, EMBED_DIM:2 * EMBED_DIM] = lax.dot_general(
        z[:, UBLK // 2:], eye, (((0,), (0,)), ((), ())),
        precision=lax.Precision.HIGHEST)


@functools.partial(jax.jit, static_argnames=("rows",))
def _relayout(table_t, rows):
    grid = (rows + UBLK - 1) // UBLK
    return pl.pallas_call(
        _relayout_body,
        out_shape=jax.ShapeDtypeStruct((grid * (UBLK // 2), 2 * EMBED_DIM),
                                       jnp.float32),
        grid=(grid,),
        in_specs=[pl.BlockSpec((EMBED_DIM, UBLK), lambda i: (0, i))],
        out_specs=pl.BlockSpec((UBLK // 2, 2 * EMBED_DIM), lambda i: (i, 0)),
        compiler_params=pltpu.CompilerParams(
            fuse_transposed_lhs_in_matmul=True),
    )(table_t)


def _sc_body(users_hbm, items_hbm, uemb_hbm, iemb_hbm, ub_hbm, ib_hbm,
             gb_hbm, out_hbm,
             uidx_v, iidx_v, uridx_v, iridx_v, urows_v, irows_v,
             ubias_v, ibias_v, gb_v, out_v, sem):
    wid = lax.axis_index("s") * NUM_CORES + lax.axis_index("c")
    base = wid * BPW

    for j in range(NCHUNK):
        pltpu.sync_copy(users_hbm.at[pl.ds(base + j * CHUNK, CHUNK)],
                        uidx_v.at[j])
        pltpu.sync_copy(items_hbm.at[pl.ds(base + j * CHUNK, CHUNK)],
                        iidx_v.at[j])
    pltpu.sync_copy(gb_hbm, gb_v)

    def ridx_body(t, _):
        j = t // (CHUNK // LANES)
        o = (t % (CHUNK // LANES)) * LANES
        u16 = uidx_v[j, pl.ds(o, LANES)]
        i16 = iidx_v[j, pl.ds(o, LANES)]
        uridx_v[j, pl.ds(o, LANES)] = \
            ((u16 >> SEG_SH) << HALF_SH) | (u16 & (HALF - 1))
        iridx_v[j, pl.ds(o, LANES)] = \
            ((i16 >> SEG_SH) << HALF_SH) | (i16 & (HALF - 1))
        return 0

    lax.fori_loop(0, NCHUNK * (CHUNK // LANES), ridx_body, 0)

    bias_copies = []
    for j in range(NCHUNK):
        sl = pl.ds(j * CHUNK, CHUNK)
        bias_copies.append(pltpu.async_copy(ub_hbm.at[uidx_v.at[j]],
                                            ubias_v.at[sl], sem))
        bias_copies.append(pltpu.async_copy(ib_hbm.at[iidx_v.at[j]],
                                            ibias_v.at[sl], sem))

    iota = jnp.arange(LANES, dtype=jnp.int32)
    gbias = gb_v[...]

    for h in range(2):
        copies = []
        for jj in range(NCHUNK // 2):
            j = h * (NCHUNK // 2) + jj
            sl = pl.ds(jj * CHUNK, CHUNK)
            copies.append(pltpu.async_copy(uemb_hbm.at[uridx_v.at[j]],
                                           urows_v.at[sl], sem))
            copies.append(pltpu.async_copy(iemb_hbm.at[iridx_v.at[j]],
                                           irows_v.at[sl], sem))
        for c in copies:
            c.wait()

        def group_body(g, _):
            rows = g * LANES + iota
            gb_off = h * (BPW // 2)
            j = (gb_off + g * LANES) // CHUNK
            o = (gb_off + g * LANES) % CHUNK
            ucol = ((uidx_v[j, pl.ds(o, LANES)] >> HALF_SH) & 1) << 6
            icol = ((iidx_v[j, pl.ds(o, LANES)] >> HALF_SH) & 1) << 6
            acc = jnp.zeros((LANES,), dtype=jnp.float32)
            for d in range(EMBED_DIM):
                cu = plsc.load_gather(urows_v, [rows, ucol + d])
                ci = plsc.load_gather(irows_v, [rows, icol + d])
                acc = acc + cu * ci
            ub = ubias_v[pl.ds(gb_off + g * LANES, LANES)]
            ib = ibias_v[pl.ds(gb_off + g * LANES, LANES)]
            out_v[pl.ds(gb_off + g * LANES, LANES)] = acc + ub + ib + gbias
            return 0

        lax.fori_loop(0, HALF_GROUPS, group_body, 0)

    for c in bias_copies:
        c.wait()
    pltpu.sync_copy(out_v, out_hbm.at[pl.ds(base, BPW)])


@jax.jit
def _bpr_scores(users, items, uemb2, iemb2, user_bias, item_bias, gb16):
    mesh = plsc.VectorSubcoreMesh(core_axis_name="c", subcore_axis_name="s",
                                  num_cores=NUM_CORES,
                                  num_subcores=NUM_SUBCORES)
    f = pl.kernel(
        _sc_body,
        out_type=jax.ShapeDtypeStruct((BATCH,), jnp.float32),
        mesh=mesh,
        compiler_params=pltpu.CompilerParams(needs_layout_passes=False,
                                             use_tc_tiling_on_sc=False),
        scratch_types=[
            pltpu.VMEM((NCHUNK, CHUNK), jnp.int32),
            pltpu.VMEM((NCHUNK, CHUNK), jnp.int32),
            pltpu.VMEM((NCHUNK, CHUNK), jnp.int32),
            pltpu.VMEM((NCHUNK, CHUNK), jnp.int32),
            pltpu.VMEM((BPW // 2, 2 * EMBED_DIM), jnp.float32),
            pltpu.VMEM((BPW // 2, 2 * EMBED_DIM), jnp.float32),
            pltpu.VMEM((BPW,), jnp.float32),
            pltpu.VMEM((BPW,), jnp.float32),
            pltpu.VMEM((LANES,), jnp.float32),
            pltpu.VMEM((BPW,), jnp.float32),
            pltpu.SemaphoreType.DMA,
        ],
    )
    return f(users, items, uemb2, iemb2, user_bias, item_bias, gb16)


def kernel(users, items, user_emb_w, item_emb_w, user_bias_w, item_bias_w,
           global_bias):
    users = users.astype(jnp.int32)
    items = items.astype(jnp.int32)
    uemb2 = _relayout(user_emb_w.T, rows=user_emb_w.shape[0])
    iemb2 = _relayout(item_emb_w.T, rows=item_emb_w.shape[0])
    gb16 = jnp.broadcast_to(global_bias.reshape(()), (16,))
    return _bpr_scores(users, items, uemb2, iemb2,
                       user_bias_w.reshape(-1), item_bias_w.reshape(-1),
                       gb16)


# speedup vs baseline: 2.0587x; 1.1343x over previous
import functools

import jax
import jax.numpy as jnp
from jax import lax
from jax.experimental import pallas as pl
from jax.experimental.pallas import tpu as pltpu
from jax.experimental.pallas import tpu_sc as plsc

BATCH = 16384
EMBED_DIM = 64
NUM_CORES = 2
NUM_SUBCORES = 16
NUM_WORKERS = NUM_CORES * NUM_SUBCORES
BPW = BATCH // NUM_WORKERS
CHUNK = 128
NCHUNK = BPW // CHUNK
LANES = 16
NGROUP = BPW // LANES


def _sc_body(users_hbm, items_hbm, uemb_hbm, iemb_hbm, ub_hbm, ib_hbm,
             gb_hbm, out_hbm,
             uidx_v, iidx_v, urows_v, irows_v, ubias_v, ibias_v, gb_v,
             out_v, sem):
    wid = lax.axis_index("s") * NUM_CORES + lax.axis_index("c")
    base = wid * BPW

    for j in range(NCHUNK):
        pltpu.sync_copy(users_hbm.at[pl.ds(base + j * CHUNK, CHUNK)],
                        uidx_v.at[j])
        pltpu.sync_copy(items_hbm.at[pl.ds(base + j * CHUNK, CHUNK)],
                        iidx_v.at[j])
    pltpu.sync_copy(gb_hbm, gb_v)

    copies = []
    for j in range(NCHUNK):
        sl = pl.ds(j * CHUNK, CHUNK)
        copies.append(pltpu.async_copy(uemb_hbm.at[uidx_v.at[j]],
                                       urows_v.at[sl], sem))
        copies.append(pltpu.async_copy(iemb_hbm.at[iidx_v.at[j]],
                                       irows_v.at[sl], sem))
        copies.append(pltpu.async_copy(ub_hbm.at[uidx_v.at[j]],
                                       ubias_v.at[sl], sem))
        copies.append(pltpu.async_copy(ib_hbm.at[iidx_v.at[j]],
                                       ibias_v.at[sl], sem))
    for c in copies:
        c.wait()

    iota = jnp.arange(LANES, dtype=jnp.int32)
    zeros_i = jnp.zeros((LANES,), dtype=jnp.int32)
    gbias = gb_v[...]

    def group_body(g, _):
        rows = g * LANES + iota
        acc = jnp.zeros((LANES,), dtype=jnp.float32)
        for d in range(EMBED_DIM):
            col = jnp.full((LANES,), d, dtype=jnp.int32)
            cu = plsc.load_gather(urows_v, [rows, col])
            ci = plsc.load_gather(irows_v, [rows, col])
            acc = acc + cu * ci
        ub = ubias_v[pl.ds(g * LANES, LANES)]
        ib = ibias_v[pl.ds(g * LANES, LANES)]
        out_v[pl.ds(g * LANES, LANES)] = acc + ub + ib + gbias
        return 0

    lax.fori_loop(0, NGROUP, group_body, 0)

    pltpu.sync_copy(out_v, out_hbm.at[pl.ds(base, BPW)])


@jax.jit
def _bpr_scores(users, items, user_emb_w, item_emb_w, user_bias_w,
                item_bias_w, global_bias):
    mesh = plsc.VectorSubcoreMesh(core_axis_name="c", subcore_axis_name="s",
                                  num_cores=NUM_CORES,
                                  num_subcores=NUM_SUBCORES)
    f = pl.kernel(
        _sc_body,
        out_type=jax.ShapeDtypeStruct((BATCH,), jnp.float32),
        mesh=mesh,
        compiler_params=pltpu.CompilerParams(needs_layout_passes=False,
                                             use_tc_tiling_on_sc=False),
        scratch_types=[
            pltpu.VMEM((NCHUNK, CHUNK), jnp.int32),
            pltpu.VMEM((NCHUNK, CHUNK), jnp.int32),
            pltpu.VMEM((BPW, EMBED_DIM), jnp.float32),
            pltpu.VMEM((BPW, EMBED_DIM), jnp.float32),
            pltpu.VMEM((BPW,), jnp.float32),
            pltpu.VMEM((BPW,), jnp.float32),
            pltpu.VMEM((LANES,), jnp.float32),
            pltpu.VMEM((BPW,), jnp.float32),
            pltpu.SemaphoreType.DMA,
        ],
    )
    return f(users, items, user_emb_w, item_emb_w, user_bias_w, item_bias_w,
             global_bias)


def kernel(users, items, user_emb_w, item_emb_w, user_bias_w, item_bias_w,
           global_bias):
    users = users.astype(jnp.int32)
    items = items.astype(jnp.int32)
    gb16 = jnp.broadcast_to(global_bias.reshape(()), (16,))
    return _bpr_scores(users, items, user_emb_w, item_emb_w,
                       user_bias_w.reshape(-1), item_bias_w.reshape(-1),
                       gb16)
